# baseline (speedup 1.0000x reference)
import jax
import jax.numpy as jnp
import numpy as np
from jax.experimental import pallas as pl
from jax.experimental.pallas import tpu as pltpu

N_Q = 8
BINS = 1024
DIM = 128
B = 16
T = 2048
BM = 2048


NH = 1
BMH = BM // NH


def _rvq_kernel(x_ref, cb_ref, quant_ref, codes_ref, loss_ref,
                cbm2_ref, d0_ref, d1_ref, d2_ref, d3_ref, cnorm_ref):
    @pl.when(pl.program_id(0) == 0)
    def _init():
        loss_ref[...] = jnp.zeros_like(loss_ref)
        for i in range(N_Q):
            cb = cb_ref[i]
            cbm2_ref[i] = (-2.0 * cb).astype(jnp.bfloat16)
            cnorm_ref[i] = jnp.sum(cb * cb, axis=1, keepdims=True)
            bits = jax.lax.bitcast_convert_type(cb, jnp.int32)
            d0_ref[i] = ((bits & 255) - 128).astype(jnp.int8)
            d1_ref[i] = (((bits >> 8) & 255) - 128).astype(jnp.int8)
            d2_ref[i] = (((bits >> 16) & 255) - 128).astype(jnp.int8)
            d3_ref[i] = (bits >> 24).astype(jnp.int8)

    rs = [x_ref[0][:, h * BMH:(h + 1) * BMH] for h in range(NH)]
    qsums = [jnp.zeros_like(rs[0]) for _ in range(NH)]
    idx_rows = [[] for _ in range(NH)]
    losses = [[] for _ in range(NH)]
    for i in range(N_Q):
        for h in range(NH):
            r = rs[h]
            rnorm = jnp.sum(r * r, axis=0, keepdims=True)
            if i > 0:
                losses[h].append(jnp.sum(rnorm))
            scores2 = jax.lax.dot_general(
                cbm2_ref[i], r.astype(jnp.bfloat16),
                (((1,), (0,)), ((), ())),
                preferred_element_type=jnp.float32)
            dist = rnorm + scores2 + cnorm_ref[i]
            idx = jnp.argmin(dist, axis=0).reshape(1, BMH)
            onehot = (jax.lax.broadcasted_iota(jnp.int32, (BINS, BMH), 0)
                      == idx).astype(jnp.int8)
            dn = (((0,), (0,)), ((), ()))
            g0 = jax.lax.dot_general(d0_ref[i], onehot, dn,
                                     preferred_element_type=jnp.int32)
            g1 = jax.lax.dot_general(d1_ref[i], onehot, dn,
                                     preferred_element_type=jnp.int32)
            g2 = jax.lax.dot_general(d2_ref[i], onehot, dn,
                                     preferred_element_type=jnp.int32)
            g3 = jax.lax.dot_general(d3_ref[i], onehot, dn,
                                     preferred_element_type=jnp.int32)
            qbits = ((g0 + 128) + ((g1 + 128) << 8)
                     + ((g2 + 128) << 16) + (g3 << 24))
            quant = jax.lax.bitcast_convert_type(qbits, jnp.float32)
            idx_rows[h].append(idx)
            rs[h] = r - quant
            qsums[h] = qsums[h] + quant
    for h in range(NH):
        losses[h].append(jnp.sum(rs[h] * rs[h]))
    for i in range(N_Q):
        tot = losses[0][i]
        for h in range(1, NH):
            tot = tot + losses[h][i]
        loss_ref[i, :] = loss_ref[i, :] + tot * (1.0 / DIM)
    quant_ref[0] = jnp.concatenate(qsums, axis=1)
    codes_ref[...] = jnp.concatenate(
        [jnp.concatenate(idx_rows[h], axis=0) for h in range(NH)], axis=1)


def kernel(x, codebooks, frame_rate):
    n_blk_t = T // BM
    grid = (B * n_blk_t,)

    quant, codes, loss = pl.pallas_call(
        _rvq_kernel,
        grid=grid,
        in_specs=[
            pl.BlockSpec((1, DIM, BM),
                         lambda p: (p // n_blk_t, 0, p % n_blk_t)),
            pl.BlockSpec((N_Q, BINS, DIM), lambda p: (0, 0, 0)),
        ],
        out_specs=[
            pl.BlockSpec((1, DIM, BM),
                         lambda p: (p // n_blk_t, 0, p % n_blk_t)),
            pl.BlockSpec((N_Q, BM), lambda p: (0, p)),
            pl.BlockSpec((N_Q, DIM), lambda p: (0, 0)),
        ],
        out_shape=[
            jax.ShapeDtypeStruct((B, DIM, T), jnp.float32),
            jax.ShapeDtypeStruct((N_Q, B * T), jnp.int32),
            jax.ShapeDtypeStruct((N_Q, DIM), jnp.float32),
        ],
        scratch_shapes=[
            pltpu.VMEM((N_Q, BINS, DIM), jnp.bfloat16),
            pltpu.VMEM((N_Q, BINS, DIM), jnp.int8),
            pltpu.VMEM((N_Q, BINS, DIM), jnp.int8),
            pltpu.VMEM((N_Q, BINS, DIM), jnp.int8),
            pltpu.VMEM((N_Q, BINS, DIM), jnp.int8),
            pltpu.VMEM((N_Q, BINS, 1), jnp.float32),
        ],
    )(x, codebooks)

    codes = codes.reshape(N_Q, B, T)
    commit_loss = jnp.sum(loss, axis=1) / (B * T * DIM)
    penalty = jnp.mean(commit_loss)
    bw = jnp.asarray(N_Q * np.log2(BINS) * frame_rate, dtype=x.dtype)
    return quant, codes, bw, penalty


# speedup vs baseline: 1.8972x; 1.8972x over previous
import jax
import jax.numpy as jnp
import numpy as np
from jax.experimental import pallas as pl
from jax.experimental.pallas import tpu as pltpu

N_Q = 8
BINS = 1024
DIM = 128
B = 16
T = 2048
BM = 2048


NH = 1
BMH = BM // NH


def _rvq_kernel(x_ref, cb_ref, quant_ref, codes_ref, loss_ref,
                cbm2_ref, hi_ref, mid_ref, lo_ref, cnorm_ref):
    @pl.when(pl.program_id(0) == 0)
    def _init():
        loss_ref[...] = jnp.zeros_like(loss_ref)
        for i in range(N_Q):
            cb = cb_ref[i]
            cbm2_ref[i] = (-2.0 * cb).astype(jnp.bfloat16)
            cnorm_ref[i] = jnp.sum(cb * cb, axis=1, keepdims=True)
            hi = cb.astype(jnp.bfloat16)
            r1 = cb - hi.astype(jnp.float32)
            mid = r1.astype(jnp.bfloat16)
            lo = (r1 - mid.astype(jnp.float32)).astype(jnp.bfloat16)
            hi_ref[i] = hi
            mid_ref[i] = mid
            lo_ref[i] = lo

    rs = [x_ref[0][:, h * BMH:(h + 1) * BMH] for h in range(NH)]
    qsums = [jnp.zeros_like(rs[0]) for _ in range(NH)]
    idx_rows = [[] for _ in range(NH)]
    losses = [[] for _ in range(NH)]
    for i in range(N_Q):
        for h in range(NH):
            r = rs[h]
            rnorm = jnp.sum(r * r, axis=0, keepdims=True)
            if i > 0:
                losses[h].append(jnp.sum(rnorm))
            scores2 = jax.lax.dot_general(
                cbm2_ref[i], r.astype(jnp.bfloat16),
                (((1,), (0,)), ((), ())),
                preferred_element_type=jnp.float32)
            dist = rnorm + scores2 + cnorm_ref[i]
            idx = jnp.argmin(dist, axis=0).reshape(1, BMH)
            onehot = (jax.lax.broadcasted_iota(jnp.int32, (BINS, BMH), 0)
                      == idx).astype(jnp.bfloat16)
            dn = (((0,), (0,)), ((), ()))
            quant = (jax.lax.dot_general(hi_ref[i], onehot, dn,
                                         preferred_element_type=jnp.float32)
                     + jax.lax.dot_general(mid_ref[i], onehot, dn,
                                           preferred_element_type=jnp.float32)
                     + jax.lax.dot_general(lo_ref[i], onehot, dn,
                                           preferred_element_type=jnp.float32))
            idx_rows[h].append(idx)
            rs[h] = r - quant
            qsums[h] = qsums[h] + quant
    for h in range(NH):
        losses[h].append(jnp.sum(rs[h] * rs[h]))
    for i in range(N_Q):
        tot = losses[0][i]
        for h in range(1, NH):
            tot = tot + losses[h][i]
        loss_ref[i, :] = loss_ref[i, :] + tot * (1.0 / DIM)
    quant_ref[0] = jnp.concatenate(qsums, axis=1)
    codes_ref[...] = jnp.concatenate(
        [jnp.concatenate(idx_rows[h], axis=0) for h in range(NH)], axis=1)


def kernel(x, codebooks, frame_rate):
    n_blk_t = T // BM
    grid = (B * n_blk_t,)

    quant, codes, loss = pl.pallas_call(
        _rvq_kernel,
        grid=grid,
        in_specs=[
            pl.BlockSpec((1, DIM, BM),
                         lambda p: (p // n_blk_t, 0, p % n_blk_t)),
            pl.BlockSpec((N_Q, BINS, DIM), lambda p: (0, 0, 0)),
        ],
        out_specs=[
            pl.BlockSpec((1, DIM, BM),
                         lambda p: (p // n_blk_t, 0, p % n_blk_t)),
            pl.BlockSpec((N_Q, BM), lambda p: (0, p)),
            pl.BlockSpec((N_Q, DIM), lambda p: (0, 0)),
        ],
        out_shape=[
            jax.ShapeDtypeStruct((B, DIM, T), jnp.float32),
            jax.ShapeDtypeStruct((N_Q, B * T), jnp.int32),
            jax.ShapeDtypeStruct((N_Q, DIM), jnp.float32),
        ],
        scratch_shapes=[
            pltpu.VMEM((N_Q, BINS, DIM), jnp.bfloat16),
            pltpu.VMEM((N_Q, BINS, DIM), jnp.bfloat16),
            pltpu.VMEM((N_Q, BINS, DIM), jnp.bfloat16),
            pltpu.VMEM((N_Q, BINS, DIM), jnp.bfloat16),
            pltpu.VMEM((N_Q, BINS, 1), jnp.float32),
        ],
    )(x, codebooks)

    codes = codes.reshape(N_Q, B, T)
    commit_loss = jnp.sum(loss, axis=1) / (B * T * DIM)
    penalty = jnp.mean(commit_loss)
    bw = jnp.asarray(N_Q * np.log2(BINS) * frame_rate, dtype=x.dtype)
    return quant, codes, bw, penalty
